# Initial kernel scaffold; baseline (speedup 1.0000x reference)
import functools

import jax
import jax.numpy as jnp
from jax import lax
from jax.experimental import pallas as pl
from jax.experimental.pallas import tpu as pltpu
from jax.experimental.pallas import tpu_sc as plsc

N = 10000
E = 320000
D_IN = 128
H = 64
G = 128

NC = 2
NS = 16
C = 80
CHUNKS = E // C
CPT = CHUNKS // (NC * NS)
RPW = N // NS

BR = 1000
NB = N // BR


def _sc_degree(dst2d):
    mesh = plsc.VectorSubcoreMesh(core_axis_name="c", subcore_axis_name="s")

    @functools.partial(
        pl.kernel,
        out_type=jax.ShapeDtypeStruct((NC, N, 16), jnp.float32),
        mesh=mesh,
        scratch_types=[
            pltpu.VMEM((CPT, C), jnp.int32),
            pltpu.VMEM((C, 16), jnp.float32),
            pltpu.VMEM((RPW, 16), jnp.float32),
            pltpu.VMEM_SHARED((N, 16), jnp.float32),
            pltpu.SemaphoreType.DMA,
        ],
    )
    def body(dst_hbm, out_hbm, dstv, ones, wb, acc, sem):
        c = lax.axis_index("c")
        s = lax.axis_index("s")
        t = c * NS + s

        def fill(i, _):
            wb[i, :] = jnp.zeros((16,), jnp.float32)
            return 0
        lax.fori_loop(0, RPW, fill, 0)

        def fill1(i, _):
            ones[i, :] = jnp.ones((16,), jnp.float32)
            return 0
        lax.fori_loop(0, C, fill1, 0)

        pltpu.sync_copy(wb, acc.at[pl.ds(s * RPW, RPW)])
        plsc.subcore_barrier()

        pltpu.sync_copy(dst_hbm.at[pl.ds(t * CPT, CPT)], dstv)

        def fire(j, _):
            pltpu.async_copy(ones, acc.at[dstv.at[j]], sem, add=True)
            return 0
        lax.fori_loop(0, CPT, fire, 0)

        def drain(j, _):
            pltpu.make_async_copy(ones, acc.at[dstv.at[j]], sem).wait()
            return 0
        lax.fori_loop(0, CPT, drain, 0)

        plsc.subcore_barrier()
        pltpu.sync_copy(acc.at[pl.ds(s * RPW, RPW)], wb)
        pltpu.sync_copy(wb, out_hbm.at[c, pl.ds(s * RPW, RPW)])

    return body(dst2d)


def _sc_scatter(p, src2d, dst2d):
    mesh = plsc.VectorSubcoreMesh(core_axis_name="c", subcore_axis_name="s")

    @functools.partial(
        pl.kernel,
        out_type=jax.ShapeDtypeStruct((NC, N, H), jnp.float32),
        mesh=mesh,
        scratch_types=[
            pltpu.VMEM((CPT, C), jnp.int32),
            pltpu.VMEM((CPT, C), jnp.int32),
            pltpu.VMEM((C, H), jnp.float32),
            pltpu.VMEM((C, H), jnp.float32),
            pltpu.VMEM((RPW, H), jnp.float32),
            pltpu.VMEM_SHARED((N, H), jnp.float32),
            pltpu.SemaphoreType.DMA,
            pltpu.SemaphoreType.DMA,
        ],
    )
    def body(p_hbm, src_hbm, dst_hbm, out_hbm,
             srcv, dstv, buf0, buf1, wb, acc, sem0, sem1):
        c = lax.axis_index("c")
        s = lax.axis_index("s")
        t = c * NS + s

        def fill(i, _):
            for j in range(H // 16):
                wb[i, pl.ds(j * 16, 16)] = jnp.zeros((16,), jnp.float32)
            return 0
        lax.fori_loop(0, RPW, fill, 0)

        pltpu.sync_copy(wb, acc.at[pl.ds(s * RPW, RPW)])
        plsc.subcore_barrier()

        pltpu.sync_copy(src_hbm.at[pl.ds(t * CPT, CPT)], srcv)
        pltpu.sync_copy(dst_hbm.at[pl.ds(t * CPT, CPT)], dstv)

        pltpu.async_copy(p_hbm.at[srcv.at[0]], buf0, sem0)
        pltpu.async_copy(p_hbm.at[srcv.at[1]], buf1, sem1)

        def step(i, _):
            jj = 2 * i
            pltpu.make_async_copy(p_hbm.at[srcv.at[jj]], buf0, sem0).wait()
            pltpu.sync_copy(buf0, acc.at[dstv.at[jj]], add=True)

            @pl.when(jj + 2 < CPT)
            def _():
                pltpu.async_copy(p_hbm.at[srcv.at[jj + 2]], buf0, sem0)

            pltpu.make_async_copy(p_hbm.at[srcv.at[jj + 1]], buf1, sem1).wait()
            pltpu.sync_copy(buf1, acc.at[dstv.at[jj + 1]], add=True)

            @pl.when(jj + 3 < CPT)
            def _():
                pltpu.async_copy(p_hbm.at[srcv.at[jj + 3]], buf1, sem1)
            return 0

        lax.fori_loop(0, CPT // 2, step, 0)
        pltpu.make_async_copy(p_hbm.at[srcv.at[CPT - 1]], buf0, sem0).wait()
        pltpu.sync_copy(buf0, acc.at[dstv.at[CPT - 1]], add=True)

        plsc.subcore_barrier()
        pltpu.sync_copy(acc.at[pl.ds(s * RPW, RPW)], wb)
        pltpu.sync_copy(wb, out_hbm.at[c, pl.ds(s * RPW, RPW)])

    return body(p, src2d, dst2d)


def _dot_t(a, w):
    return lax.dot_general(a, w, (((1,), (1,)), ((), ())),
                           preferred_element_type=jnp.float32)


def _tc_first(x, W0, dacc):
    def body(x_ref, w_ref, d_ref, p_ref, dis_ref):
        deg = d_ref[0] + d_ref[1] + 1.0
        dis = lax.rsqrt(deg)
        dis_ref[...] = dis
        p_ref[...] = _dot_t(x_ref[...], w_ref[...]) * dis[:, 0:1]

    return pl.pallas_call(
        body,
        grid=(NB,),
        in_specs=[
            pl.BlockSpec((BR, D_IN), lambda i: (i, 0)),
            pl.BlockSpec((H, D_IN), lambda i: (0, 0)),
            pl.BlockSpec((NC, BR, 16), lambda i: (0, i, 0)),
        ],
        out_specs=[
            pl.BlockSpec((BR, H), lambda i: (i, 0)),
            pl.BlockSpec((BR, 16), lambda i: (i, 0)),
        ],
        out_shape=[
            jax.ShapeDtypeStruct((N, H), jnp.float32),
            jax.ShapeDtypeStruct((N, 16), jnp.float32),
        ],
    )(x, W0, dacc)


def _tc_stats(sacc, p, b, dis16):
    def body(s_ref, p_ref, b_ref, dis_ref, c_ref, st_ref):
        i = pl.program_id(0)
        cblk = dis_ref[:, 0:1] * (s_ref[0] + s_ref[1] + p_ref[...]) + b_ref[...]
        c_ref[...] = cblk
        ssum = jnp.sum(cblk, axis=0, keepdims=True)
        ssq = jnp.sum(cblk * cblk, axis=0, keepdims=True)
        blk = jnp.concatenate([ssum, ssq], axis=0)

        @pl.when(i == 0)
        def _():
            st_ref[...] = blk

        @pl.when(i > 0)
        def _():
            st_ref[...] += blk

    return pl.pallas_call(
        body,
        grid=(NB,),
        in_specs=[
            pl.BlockSpec((NC, BR, H), lambda i: (0, i, 0)),
            pl.BlockSpec((BR, H), lambda i: (i, 0)),
            pl.BlockSpec((1, H), lambda i: (0, 0)),
            pl.BlockSpec((BR, 16), lambda i: (i, 0)),
        ],
        out_specs=[
            pl.BlockSpec((BR, H), lambda i: (i, 0)),
            pl.BlockSpec((2, H), lambda i: (0, 0)),
        ],
        out_shape=[
            jax.ShapeDtypeStruct((N, H), jnp.float32),
            jax.ShapeDtypeStruct((2, H), jnp.float32),
        ],
    )(sacc, p, b, dis16)


def _tc_apply(carr, stats, g, t, res, Wn, dis16):
    has_res = res is not None

    def body(*refs):
        if has_res:
            c_ref, st_ref, g_ref, t_ref, r_ref, w_ref, dis_ref, h_ref, p_ref = refs
        else:
            c_ref, st_ref, g_ref, t_ref, w_ref, dis_ref, h_ref, p_ref = refs
        mu = st_ref[0:1] / N
        var = st_ref[1:2] / N - mu * mu
        scale = lax.rsqrt(var + 1e-5) * g_ref[...]
        h = jnp.maximum((c_ref[...] - mu) * scale + t_ref[...], 0.0)
        if has_res:
            h = h + r_ref[...]
        h_ref[...] = h
        p_ref[...] = _dot_t(h, w_ref[...]) * dis_ref[:, 0:1]

    in_arrays = [carr, stats, g, t] + ([res] if has_res else []) + [Wn, dis16]
    in_specs = [
        pl.BlockSpec((BR, H), lambda i: (i, 0)),
        pl.BlockSpec((2, H), lambda i: (0, 0)),
        pl.BlockSpec((1, H), lambda i: (0, 0)),
        pl.BlockSpec((1, H), lambda i: (0, 0)),
    ] + ([pl.BlockSpec((BR, H), lambda i: (i, 0))] if has_res else []) + [
        pl.BlockSpec((H, H), lambda i: (0, 0)),
        pl.BlockSpec((BR, 16), lambda i: (i, 0)),
    ]
    return pl.pallas_call(
        body,
        grid=(NB,),
        in_specs=in_specs,
        out_specs=[
            pl.BlockSpec((BR, H), lambda i: (i, 0)),
            pl.BlockSpec((BR, H), lambda i: (i, 0)),
        ],
        out_shape=[
            jax.ShapeDtypeStruct((N, H), jnp.float32),
            jax.ShapeDtypeStruct((N, H), jnp.float32),
        ],
    )(*in_arrays)


def _tc_final(carr, stats, g, t, res, batch2d, lw0, lb0, lw1, lb1):
    def body(c_ref, st_ref, g_ref, t_ref, r_ref, b_ref,
             w0_ref, b0_ref, w1_ref, b1_ref, o_ref, pool_acc, cnt_acc):
        i = pl.program_id(0)
        mu = st_ref[0:1] / N
        var = st_ref[1:2] / N - mu * mu
        scale = lax.rsqrt(var + 1e-5) * g_ref[...]
        h = jnp.maximum((c_ref[...] - mu) * scale + t_ref[...], 0.0) + r_ref[...]

        iota = lax.broadcasted_iota(jnp.int32, (1, G), 1)
        mask = (b_ref[...] == iota).astype(jnp.float32)
        pool_blk = lax.dot_general(mask, h, (((0,), (0,)), ((), ())),
                                   preferred_element_type=jnp.float32)
        cnt_blk = lax.dot_general(mask, jnp.ones((BR, 8), jnp.float32),
                                  (((0,), (0,)), ((), ())),
                                  preferred_element_type=jnp.float32)

        @pl.when(i == 0)
        def _():
            pool_acc[...] = pool_blk
            cnt_acc[...] = cnt_blk

        @pl.when(i > 0)
        def _():
            pool_acc[...] += pool_blk
            cnt_acc[...] += cnt_blk

        @pl.when(i == NB - 1)
        def _():
            cnt = jnp.maximum(cnt_acc[:, 0:1], 1.0)
            hp = pool_acc[...] / cnt
            z = jnp.maximum(_dot_t(hp, w0_ref[...]) + b0_ref[...], 0.0)
            o_ref[...] = _dot_t(z, w1_ref[...]) + b1_ref[...]

    return pl.pallas_call(
        body,
        grid=(NB,),
        in_specs=[
            pl.BlockSpec((BR, H), lambda i: (i, 0)),
            pl.BlockSpec((2, H), lambda i: (0, 0)),
            pl.BlockSpec((1, H), lambda i: (0, 0)),
            pl.BlockSpec((1, H), lambda i: (0, 0)),
            pl.BlockSpec((BR, H), lambda i: (i, 0)),
            pl.BlockSpec((BR, 1), lambda i: (i, 0)),
            pl.BlockSpec((H, H), lambda i: (0, 0)),
            pl.BlockSpec((1, H), lambda i: (0, 0)),
            pl.BlockSpec((1, H), lambda i: (0, 0)),
            pl.BlockSpec((1, 1), lambda i: (0, 0)),
        ],
        out_specs=pl.BlockSpec((G, 1), lambda i: (0, 0)),
        out_shape=jax.ShapeDtypeStruct((G, 1), jnp.float32),
        scratch_shapes=[
            pltpu.VMEM((G, H), jnp.float32),
            pltpu.VMEM((G, 8), jnp.float32),
        ],
    )(carr, stats, g, t, res, batch2d, lw0, lb0, lw1, lb1)


def kernel(x, edge_index, batch, W0, b0, W1, b1, W2, b2,
           g0, t0, g1, t1, g2, t2, lw0, lb0, lw1, lb1):
    src2d = edge_index[0].astype(jnp.int32).reshape(CHUNKS, C)
    dst2d = edge_index[1].astype(jnp.int32).reshape(CHUNKS, C)
    batch2d = batch.astype(jnp.int32).reshape(N, 1)

    b0r = b0.reshape(1, H); b1r = b1.reshape(1, H); b2r = b2.reshape(1, H)
    g0r = g0.reshape(1, H); g1r = g1.reshape(1, H); g2r = g2.reshape(1, H)
    t0r = t0.reshape(1, H); t1r = t1.reshape(1, H); t2r = t2.reshape(1, H)
    lb0r = lb0.reshape(1, H); lb1r = lb1.reshape(1, 1)

    dacc = _sc_degree(dst2d)
    p1, dis16 = _tc_first(x, W0, dacc)

    s1 = _sc_scatter(p1, src2d, dst2d)
    c1, st1 = _tc_stats(s1, p1, b0r, dis16)
    h1, p2 = _tc_apply(c1, st1, g0r, t0r, None, W1, dis16)

    s2 = _sc_scatter(p2, src2d, dst2d)
    c2, st2 = _tc_stats(s2, p2, b1r, dis16)
    h2, p3 = _tc_apply(c2, st2, g1r, t1r, h1, W2, dis16)

    s3 = _sc_scatter(p3, src2d, dst2d)
    c3, st3 = _tc_stats(s3, p3, b2r, dis16)
    out = _tc_final(c3, st3, g2r, t2r, h2, batch2d, lw0, lb0r, lw1, lb1r)

    return out.reshape(G)


# speedup vs baseline: 28.7443x; 28.7443x over previous
import functools

import jax
import jax.numpy as jnp
from jax import lax
from jax.experimental import pallas as pl
from jax.experimental.pallas import tpu as pltpu
from jax.experimental.pallas import tpu_sc as plsc

N = 10000
E = 320000
D_IN = 128
H = 64
G = 128

NC = 2
NS = 16
C = 80
CHUNKS = E // C
CPT = CHUNKS // (NC * NS)
RPW = N // NS

BR = 1000
NB = N // BR


def _sc_degree(dst2d):
    mesh = plsc.VectorSubcoreMesh(core_axis_name="c", subcore_axis_name="s")

    @functools.partial(
        pl.kernel,
        out_type=jax.ShapeDtypeStruct((NC, NS, RPW, 16), jnp.float32),
        mesh=mesh,
        scratch_types=[
            pltpu.VMEM((CPT, C), jnp.int32),
            pltpu.VMEM((C, 16), jnp.float32),
            pltpu.VMEM((RPW, 16), jnp.float32),
            pltpu.VMEM_SHARED((N, 16), jnp.float32),
            pltpu.SemaphoreType.DMA,
        ],
        compiler_params=pltpu.CompilerParams(use_tc_tiling_on_sc=False),
    )
    def body(dst_hbm, out_hbm, dstv, ones, wb, acc, sem):
        c = lax.axis_index("c")
        s = lax.axis_index("s")
        t = c * NS + s

        def fill(i, _):
            wb[i, :] = jnp.zeros((16,), jnp.float32)
            return 0
        lax.fori_loop(0, RPW, fill, 0)

        def fill1(i, _):
            ones[i, :] = jnp.ones((16,), jnp.float32)
            return 0
        lax.fori_loop(0, C, fill1, 0)

        pltpu.sync_copy(wb, acc.at[pl.ds(s * RPW, RPW)])
        plsc.subcore_barrier()

        pltpu.sync_copy(dst_hbm.at[t], dstv)

        def fire(j, _):
            pltpu.async_copy(ones, acc.at[dstv.at[j]], sem, add=True)
            return 0
        lax.fori_loop(0, CPT, fire, 0)

        def drain(j, _):
            pltpu.make_async_copy(ones, acc.at[dstv.at[j]], sem).wait()
            return 0
        lax.fori_loop(0, CPT, drain, 0)

        plsc.subcore_barrier()
        pltpu.sync_copy(acc.at[pl.ds(s * RPW, RPW)], wb)
        pltpu.sync_copy(wb, out_hbm.at[c, s])

    return body(dst2d)


def _sc_scatter(p, src2d, dst2d):
    mesh = plsc.VectorSubcoreMesh(core_axis_name="c", subcore_axis_name="s")

    SRO = 624
    SRN = 640

    @functools.partial(
        pl.kernel,
        out_type=jax.ShapeDtypeStruct((NC, NS, RPW, H), jnp.float32),
        mesh=mesh,
        scratch_types=[
            pltpu.VMEM((CPT, C), jnp.int32),
            pltpu.VMEM((CPT, C), jnp.int32),
            pltpu.VMEM((C, H), jnp.float32),
            pltpu.VMEM((C, H), jnp.float32),
            pltpu.VMEM((SRN, H), jnp.float32),
            pltpu.VMEM_SHARED((N, H), jnp.float32),
            pltpu.SemaphoreType.DMA,
            pltpu.SemaphoreType.DMA,
        ],
        compiler_params=pltpu.CompilerParams(use_tc_tiling_on_sc=False),
    )
    def body(p_hbm, src_hbm, dst_hbm, out_hbm,
             srcv, dstv, buf0, buf1, wb, acc, sem0, sem1):
        c = lax.axis_index("c")
        s = lax.axis_index("s")
        t = c * NS + s

        def fill(i, _):
            for j in range(H // 16):
                wb[i, pl.ds(j * 16, 16)] = jnp.zeros((16,), jnp.float32)
            return 0
        lax.fori_loop(0, SRN, fill, 0)
        pltpu.sync_copy(wb.at[pl.ds(0, RPW)], acc.at[pl.ds(s * RPW, RPW)])
        plsc.subcore_barrier()

        pltpu.sync_copy(src_hbm.at[t], srcv)
        pltpu.sync_copy(dst_hbm.at[t], dstv)

        pltpu.async_copy(p_hbm.at[srcv.at[0]], buf0, sem0)
        pltpu.async_copy(p_hbm.at[srcv.at[1]], buf1, sem1)

        def step(i, _):
            jj = 2 * i
            pltpu.make_async_copy(p_hbm.at[srcv.at[jj]], buf0, sem0).wait()
            pltpu.sync_copy(buf0, acc.at[dstv.at[jj]], add=True)

            @pl.when(jj + 2 < CPT)
            def _():
                pltpu.async_copy(p_hbm.at[srcv.at[jj + 2]], buf0, sem0)

            pltpu.make_async_copy(p_hbm.at[srcv.at[jj + 1]], buf1, sem1).wait()
            pltpu.sync_copy(buf1, acc.at[dstv.at[jj + 1]], add=True)

            @pl.when(jj + 3 < CPT)
            def _():
                pltpu.async_copy(p_hbm.at[srcv.at[jj + 3]], buf1, sem1)
            return 0

        lax.fori_loop(0, CPT // 2, step, 0)
        pltpu.make_async_copy(p_hbm.at[srcv.at[CPT - 1]], buf0, sem0).wait()
        pltpu.sync_copy(buf0, acc.at[dstv.at[CPT - 1]], add=True)

        plsc.subcore_barrier()
        pltpu.sync_copy(acc.at[pl.ds(s * RPW, RPW)], wb.at[pl.ds(0, RPW)])
        pltpu.sync_copy(wb.at[pl.ds(0, RPW)], out_hbm.at[c, s])

    return body(p, src2d, dst2d)


def _dot_t(a, w):
    return lax.dot_general(a, w, (((1,), (1,)), ((), ())),
                           preferred_element_type=jnp.float32)


def _tc_first(x, W0, dacc):
    def body(x_ref, w_ref, d_ref, p_ref, dis_ref):
        deg = d_ref[0] + d_ref[1] + 1.0
        dis = lax.rsqrt(deg)
        dis_ref[...] = dis
        p_ref[...] = _dot_t(x_ref[...], w_ref[...]) * dis[:, 0:1]

    return pl.pallas_call(
        body,
        grid=(NB,),
        in_specs=[
            pl.BlockSpec((BR, D_IN), lambda i: (i, 0)),
            pl.BlockSpec((H, D_IN), lambda i: (0, 0)),
            pl.BlockSpec((NC, BR, 16), lambda i: (0, i, 0)),
        ],
        out_specs=[
            pl.BlockSpec((BR, H), lambda i: (i, 0)),
            pl.BlockSpec((BR, 16), lambda i: (i, 0)),
        ],
        out_shape=[
            jax.ShapeDtypeStruct((N, H), jnp.float32),
            jax.ShapeDtypeStruct((N, 16), jnp.float32),
        ],
    )(x, W0, dacc)


def _tc_stats(sacc, p, b, dis16):
    def body(s_ref, p_ref, b_ref, dis_ref, c_ref, st_ref):
        i = pl.program_id(0)
        cblk = dis_ref[:, 0:1] * (s_ref[0] + s_ref[1] + p_ref[...]) + b_ref[...]
        c_ref[...] = cblk
        ssum = jnp.sum(cblk, axis=0, keepdims=True)
        ssq = jnp.sum(cblk * cblk, axis=0, keepdims=True)
        blk = jnp.concatenate([ssum, ssq], axis=0)

        @pl.when(i == 0)
        def _():
            st_ref[...] = blk

        @pl.when(i > 0)
        def _():
            st_ref[...] += blk

    return pl.pallas_call(
        body,
        grid=(NB,),
        in_specs=[
            pl.BlockSpec((NC, BR, H), lambda i: (0, i, 0)),
            pl.BlockSpec((BR, H), lambda i: (i, 0)),
            pl.BlockSpec((1, H), lambda i: (0, 0)),
            pl.BlockSpec((BR, 16), lambda i: (i, 0)),
        ],
        out_specs=[
            pl.BlockSpec((BR, H), lambda i: (i, 0)),
            pl.BlockSpec((2, H), lambda i: (0, 0)),
        ],
        out_shape=[
            jax.ShapeDtypeStruct((N, H), jnp.float32),
            jax.ShapeDtypeStruct((2, H), jnp.float32),
        ],
    )(sacc, p, b, dis16)


def _tc_apply(carr, stats, g, t, res, Wn, dis16):
    has_res = res is not None

    def body(*refs):
        if has_res:
            c_ref, st_ref, g_ref, t_ref, r_ref, w_ref, dis_ref, h_ref, p_ref = refs
        else:
            c_ref, st_ref, g_ref, t_ref, w_ref, dis_ref, h_ref, p_ref = refs
        mu = st_ref[0:1] / N
        var = st_ref[1:2] / N - mu * mu
        scale = lax.rsqrt(var + 1e-5) * g_ref[...]
        h = jnp.maximum((c_ref[...] - mu) * scale + t_ref[...], 0.0)
        if has_res:
            h = h + r_ref[...]
        h_ref[...] = h
        p_ref[...] = _dot_t(h, w_ref[...]) * dis_ref[:, 0:1]

    in_arrays = [carr, stats, g, t] + ([res] if has_res else []) + [Wn, dis16]
    in_specs = [
        pl.BlockSpec((BR, H), lambda i: (i, 0)),
        pl.BlockSpec((2, H), lambda i: (0, 0)),
        pl.BlockSpec((1, H), lambda i: (0, 0)),
        pl.BlockSpec((1, H), lambda i: (0, 0)),
    ] + ([pl.BlockSpec((BR, H), lambda i: (i, 0))] if has_res else []) + [
        pl.BlockSpec((H, H), lambda i: (0, 0)),
        pl.BlockSpec((BR, 16), lambda i: (i, 0)),
    ]
    return pl.pallas_call(
        body,
        grid=(NB,),
        in_specs=in_specs,
        out_specs=[
            pl.BlockSpec((BR, H), lambda i: (i, 0)),
            pl.BlockSpec((BR, H), lambda i: (i, 0)),
        ],
        out_shape=[
            jax.ShapeDtypeStruct((N, H), jnp.float32),
            jax.ShapeDtypeStruct((N, H), jnp.float32),
        ],
    )(*in_arrays)


def _tc_final(carr, stats, g, t, res, batch2d, lw0, lb0, lw1, lb1):
    def body(c_ref, st_ref, g_ref, t_ref, r_ref, b_ref,
             w0_ref, b0_ref, w1_ref, b1_ref, o_ref, pool_acc, cnt_acc):
        i = pl.program_id(0)
        mu = st_ref[0:1] / N
        var = st_ref[1:2] / N - mu * mu
        scale = lax.rsqrt(var + 1e-5) * g_ref[...]
        h = jnp.maximum((c_ref[...] - mu) * scale + t_ref[...], 0.0) + r_ref[...]

        iota = lax.broadcasted_iota(jnp.int32, (1, G), 1)
        mask = (b_ref[...] == iota).astype(jnp.float32)
        pool_blk = lax.dot_general(mask, h, (((0,), (0,)), ((), ())),
                                   preferred_element_type=jnp.float32)
        cnt_blk = lax.dot_general(mask, jnp.ones((BR, 8), jnp.float32),
                                  (((0,), (0,)), ((), ())),
                                  preferred_element_type=jnp.float32)

        @pl.when(i == 0)
        def _():
            pool_acc[...] = pool_blk
            cnt_acc[...] = cnt_blk

        @pl.when(i > 0)
        def _():
            pool_acc[...] += pool_blk
            cnt_acc[...] += cnt_blk

        @pl.when(i == NB - 1)
        def _():
            cnt = jnp.maximum(cnt_acc[:, 0:1], 1.0)
            hp = pool_acc[...] / cnt
            z = jnp.maximum(_dot_t(hp, w0_ref[...]) + b0_ref[...], 0.0)
            o_ref[...] = _dot_t(z, w1_ref[...]) + b1_ref[...]

    return pl.pallas_call(
        body,
        grid=(NB,),
        in_specs=[
            pl.BlockSpec((BR, H), lambda i: (i, 0)),
            pl.BlockSpec((2, H), lambda i: (0, 0)),
            pl.BlockSpec((1, H), lambda i: (0, 0)),
            pl.BlockSpec((1, H), lambda i: (0, 0)),
            pl.BlockSpec((BR, H), lambda i: (i, 0)),
            pl.BlockSpec((BR, 1), lambda i: (i, 0)),
            pl.BlockSpec((H, H), lambda i: (0, 0)),
            pl.BlockSpec((1, H), lambda i: (0, 0)),
            pl.BlockSpec((G, H), lambda i: (0, 0)),
            pl.BlockSpec((1, G), lambda i: (0, 0)),
        ],
        out_specs=pl.BlockSpec((G, G), lambda i: (0, 0)),
        out_shape=jax.ShapeDtypeStruct((G, G), jnp.float32),
        scratch_shapes=[
            pltpu.VMEM((G, H), jnp.float32),
            pltpu.VMEM((G, 8), jnp.float32),
        ],
    )(carr, stats, g, t, res, batch2d, lw0, lb0, lw1, lb1)


def kernel(x, edge_index, batch, W0, b0, W1, b1, W2, b2,
           g0, t0, g1, t1, g2, t2, lw0, lb0, lw1, lb1):
    src2d = edge_index[0].astype(jnp.int32).reshape(NC * NS, CPT, C)
    dst2d = edge_index[1].astype(jnp.int32).reshape(NC * NS, CPT, C)
    batch2d = batch.astype(jnp.int32).reshape(N, 1)

    b0r = b0.reshape(1, H); b1r = b1.reshape(1, H); b2r = b2.reshape(1, H)
    g0r = g0.reshape(1, H); g1r = g1.reshape(1, H); g2r = g2.reshape(1, H)
    t0r = t0.reshape(1, H); t1r = t1.reshape(1, H); t2r = t2.reshape(1, H)
    lb0r = lb0.reshape(1, H)
    lw1p = jnp.zeros((G, H), jnp.float32).at[0].set(lw1[0])
    lb1r = jnp.broadcast_to(lb1.reshape(1, 1), (1, G))

    dacc = _sc_degree(dst2d).reshape(NC, N, 16)
    p1, dis16 = _tc_first(x, W0, dacc)

    s1 = _sc_scatter(p1, src2d, dst2d).reshape(NC, N, H)
    c1, st1 = _tc_stats(s1, p1, b0r, dis16)
    h1, p2 = _tc_apply(c1, st1, g0r, t0r, None, W1, dis16)

    s2 = _sc_scatter(p2, src2d, dst2d).reshape(NC, N, H)
    c2, st2 = _tc_stats(s2, p2, b1r, dis16)
    h2, p3 = _tc_apply(c2, st2, g1r, t1r, h1, W2, dis16)

    s3 = _sc_scatter(p3, src2d, dst2d).reshape(NC, N, H)
    c3, st3 = _tc_stats(s3, p3, b2r, dis16)
    out = _tc_final(c3, st3, g2r, t2r, h2, batch2d, lw0, lb0r, lw1p, lb1r)

    return out[:, 0]


# speedup vs baseline: 35.2182x; 1.2252x over previous
import functools

import jax
import jax.numpy as jnp
from jax import lax
from jax.experimental import pallas as pl
from jax.experimental.pallas import tpu as pltpu
from jax.experimental.pallas import tpu_sc as plsc

N = 10000
E = 320000
D_IN = 128
H = 64
G = 128

NC = 2
NS = 16
C = 80
CHUNKS = E // C
CPT = CHUNKS // (NC * NS)
RPW = N // NS

BR = 1000
NB = N // BR


def _sc_degree(dst2d):
    mesh = plsc.VectorSubcoreMesh(core_axis_name="c", subcore_axis_name="s")

    @functools.partial(
        pl.kernel,
        out_type=jax.ShapeDtypeStruct((NC, NS, RPW, 16), jnp.float32),
        mesh=mesh,
        scratch_types=[
            pltpu.VMEM((CPT, C), jnp.int32),
            pltpu.VMEM((C, 16), jnp.float32),
            pltpu.VMEM((RPW, 16), jnp.float32),
            pltpu.VMEM_SHARED((N, 16), jnp.float32),
            pltpu.SemaphoreType.DMA,
        ],
        compiler_params=pltpu.CompilerParams(use_tc_tiling_on_sc=False),
    )
    def body(dst_hbm, out_hbm, dstv, ones, wb, acc, sem):
        c = lax.axis_index("c")
        s = lax.axis_index("s")
        t = c * NS + s

        def fill(i, _):
            wb[i, :] = jnp.zeros((16,), jnp.float32)
            return 0
        lax.fori_loop(0, RPW, fill, 0)

        def fill1(i, _):
            ones[i, :] = jnp.ones((16,), jnp.float32)
            return 0
        lax.fori_loop(0, C, fill1, 0)

        pltpu.sync_copy(wb, acc.at[pl.ds(s * RPW, RPW)])
        plsc.subcore_barrier()

        pltpu.sync_copy(dst_hbm.at[t], dstv)

        def fire(j, _):
            pltpu.async_copy(ones, acc.at[dstv.at[j]], sem, add=True)
            return 0
        lax.fori_loop(0, CPT, fire, 0)

        def drain(j, _):
            pltpu.make_async_copy(ones, acc.at[dstv.at[j]], sem).wait()
            return 0
        lax.fori_loop(0, CPT, drain, 0)

        plsc.subcore_barrier()
        pltpu.sync_copy(acc.at[pl.ds(s * RPW, RPW)], wb)
        pltpu.sync_copy(wb, out_hbm.at[c, s])

    return body(dst2d)


def _sc_scatter(p, src2d, dst2d):
    mesh = plsc.VectorSubcoreMesh(core_axis_name="c", subcore_axis_name="s")

    SRO = 624
    SRN = 640

    @functools.partial(
        pl.kernel,
        out_type=jax.ShapeDtypeStruct((NC, NS, RPW, H), jnp.float32),
        mesh=mesh,
        scratch_types=[
            pltpu.VMEM((CPT, C), jnp.int32),
            pltpu.VMEM((CPT, C), jnp.int32),
            [pltpu.VMEM((C, H), jnp.float32) for _ in range(4)],
            pltpu.VMEM((SRN, H), jnp.float32),
            pltpu.VMEM_SHARED((N, H), jnp.float32),
            [pltpu.SemaphoreType.DMA for _ in range(4)],
        ],
        compiler_params=pltpu.CompilerParams(use_tc_tiling_on_sc=False),
    )
    def body(p_hbm, src_hbm, dst_hbm, out_hbm,
             srcv, dstv, bufs, wb, acc, sems):
        c = lax.axis_index("c")
        s = lax.axis_index("s")
        t = c * NS + s

        def fill(i, _):
            for j in range(H // 16):
                wb[i, pl.ds(j * 16, 16)] = jnp.zeros((16,), jnp.float32)
            return 0
        lax.fori_loop(0, SRN, fill, 0)
        pltpu.sync_copy(wb.at[pl.ds(0, RPW)], acc.at[pl.ds(s * RPW, RPW)])
        plsc.subcore_barrier()

        pltpu.sync_copy(src_hbm.at[t], srcv)
        pltpu.sync_copy(dst_hbm.at[t], dstv)

        for b in range(4):
            pltpu.async_copy(p_hbm.at[srcv.at[b]], bufs[b], sems[b])

        def step(g, _):
            for b in range(4):
                jj = 4 * g + b
                pltpu.make_async_copy(p_hbm.at[srcv.at[jj]], bufs[b],
                                      sems[b]).wait()
                pltpu.sync_copy(bufs[b], acc.at[dstv.at[jj]], add=True)

                @pl.when(jj + 4 < CPT)
                def _():
                    pltpu.async_copy(p_hbm.at[srcv.at[jj + 4]], bufs[b],
                                     sems[b])
            return 0

        lax.fori_loop(0, CPT // 4, step, 0)
        pltpu.make_async_copy(p_hbm.at[srcv.at[CPT - 1]], bufs[0],
                              sems[0]).wait()
        pltpu.sync_copy(bufs[0], acc.at[dstv.at[CPT - 1]], add=True)

        plsc.subcore_barrier()
        pltpu.sync_copy(acc.at[pl.ds(s * RPW, RPW)], wb.at[pl.ds(0, RPW)])
        pltpu.sync_copy(wb.at[pl.ds(0, RPW)], out_hbm.at[c, s])

    return body(p, src2d, dst2d)


def _dot_t(a, w):
    return lax.dot_general(a, w, (((1,), (1,)), ((), ())),
                           preferred_element_type=jnp.float32)


def _tc_first(x, W0, dacc):
    def body(x_ref, w_ref, d_ref, p_ref, dis_ref):
        deg = d_ref[0] + d_ref[1] + 1.0
        dis = lax.rsqrt(deg)
        dis_ref[...] = dis
        p_ref[...] = _dot_t(x_ref[...], w_ref[...]) * dis[:, 0:1]

    return pl.pallas_call(
        body,
        grid=(NB,),
        in_specs=[
            pl.BlockSpec((BR, D_IN), lambda i: (i, 0)),
            pl.BlockSpec((H, D_IN), lambda i: (0, 0)),
            pl.BlockSpec((NC, BR, 16), lambda i: (0, i, 0)),
        ],
        out_specs=[
            pl.BlockSpec((BR, H), lambda i: (i, 0)),
            pl.BlockSpec((BR, 16), lambda i: (i, 0)),
        ],
        out_shape=[
            jax.ShapeDtypeStruct((N, H), jnp.float32),
            jax.ShapeDtypeStruct((N, 16), jnp.float32),
        ],
    )(x, W0, dacc)


def _tc_layer(sacc, p, b, g, t, res, Wn, dis16):
    has_res = res is not None

    def body(*refs):
        if has_res:
            (s_ref, p_ref, b_ref, g_ref, t_ref, r_ref, w_ref, dis_ref,
             h_ref, pn_ref, c_v, st_v) = refs
        else:
            (s_ref, p_ref, b_ref, g_ref, t_ref, w_ref, dis_ref,
             h_ref, pn_ref, c_v, st_v) = refs
        ph = pl.program_id(0)
        i = pl.program_id(1)

        @pl.when(ph == 0)
        def _():
            cblk = (dis_ref[:, 0:1] * (s_ref[0] + s_ref[1] + p_ref[...])
                    + b_ref[...])
            c_v[i] = cblk
            ssum = jnp.sum(cblk, axis=0, keepdims=True)
            ssq = jnp.sum(cblk * cblk, axis=0, keepdims=True)
            blk = jnp.concatenate([ssum, ssq], axis=0)

            @pl.when(i == 0)
            def _():
                st_v[...] = blk

            @pl.when(i > 0)
            def _():
                st_v[...] += blk

        @pl.when(ph == 1)
        def _():
            mu = st_v[0:1] / N
            var = st_v[1:2] / N - mu * mu
            scale = lax.rsqrt(var + 1e-5) * g_ref[...]
            h = jnp.maximum((c_v[i] - mu) * scale + t_ref[...], 0.0)
            if has_res:
                h = h + r_ref[...]
            h_ref[...] = h
            pn_ref[...] = _dot_t(h, w_ref[...]) * dis_ref[:, 0:1]

    in_arrays = [sacc, p, b, g, t] + ([res] if has_res else []) + [Wn, dis16]
    in_specs = [
        pl.BlockSpec((NC, BR, H), lambda ph, i: (0, i, 0)),
        pl.BlockSpec((BR, H), lambda ph, i: (i, 0)),
        pl.BlockSpec((1, H), lambda ph, i: (0, 0)),
        pl.BlockSpec((1, H), lambda ph, i: (0, 0)),
        pl.BlockSpec((1, H), lambda ph, i: (0, 0)),
    ] + ([pl.BlockSpec((BR, H), lambda ph, i: (i, 0))] if has_res else []) + [
        pl.BlockSpec((H, H), lambda ph, i: (0, 0)),
        pl.BlockSpec((BR, 16), lambda ph, i: (i, 0)),
    ]
    return pl.pallas_call(
        body,
        grid=(2, NB),
        in_specs=in_specs,
        out_specs=[
            pl.BlockSpec((BR, H), lambda ph, i: (i, 0)),
            pl.BlockSpec((BR, H), lambda ph, i: (i, 0)),
        ],
        out_shape=[
            jax.ShapeDtypeStruct((N, H), jnp.float32),
            jax.ShapeDtypeStruct((N, H), jnp.float32),
        ],
        scratch_shapes=[
            pltpu.VMEM((NB, BR, H), jnp.float32),
            pltpu.VMEM((2, H), jnp.float32),
        ],
    )(*in_arrays)


def _tc_final(sacc, p, b, g, t, res, batch2d, lw0, lb0, lw1, lb1, dis16):
    def body(s_ref, p_ref, bias_ref, g_ref, t_ref, r_ref, b_ref,
             w0_ref, b0_ref, w1_ref, b1_ref, dis_ref,
             o_ref, c_v, st_v, pool_acc, cnt_acc):
        ph = pl.program_id(0)
        i = pl.program_id(1)

        @pl.when(ph == 0)
        def _():
            cblk = (dis_ref[:, 0:1] * (s_ref[0] + s_ref[1] + p_ref[...])
                    + bias_ref[...])
            c_v[i] = cblk
            ssum = jnp.sum(cblk, axis=0, keepdims=True)
            ssq = jnp.sum(cblk * cblk, axis=0, keepdims=True)
            blk = jnp.concatenate([ssum, ssq], axis=0)

            @pl.when(i == 0)
            def _():
                st_v[...] = blk

            @pl.when(i > 0)
            def _():
                st_v[...] += blk

        @pl.when(ph == 1)
        def _():
            mu = st_v[0:1] / N
            var = st_v[1:2] / N - mu * mu
            scale = lax.rsqrt(var + 1e-5) * g_ref[...]
            h = (jnp.maximum((c_v[i] - mu) * scale + t_ref[...], 0.0)
                 + r_ref[...])

            iota = lax.broadcasted_iota(jnp.int32, (1, G), 1)
            mask = (b_ref[...] == iota).astype(jnp.float32)
            pool_blk = lax.dot_general(mask, h, (((0,), (0,)), ((), ())),
                                       preferred_element_type=jnp.float32)
            cnt_blk = lax.dot_general(mask, jnp.ones((BR, 8), jnp.float32),
                                      (((0,), (0,)), ((), ())),
                                      preferred_element_type=jnp.float32)

            @pl.when(i == 0)
            def _():
                pool_acc[...] = pool_blk
                cnt_acc[...] = cnt_blk

            @pl.when(i > 0)
            def _():
                pool_acc[...] += pool_blk
                cnt_acc[...] += cnt_blk

            @pl.when(i == NB - 1)
            def _():
                cnt = jnp.maximum(cnt_acc[:, 0:1], 1.0)
                hp = pool_acc[...] / cnt
                z = jnp.maximum(_dot_t(hp, w0_ref[...]) + b0_ref[...], 0.0)
                o_ref[...] = _dot_t(z, w1_ref[...]) + b1_ref[...]

    return pl.pallas_call(
        body,
        grid=(2, NB),
        in_specs=[
            pl.BlockSpec((NC, BR, H), lambda ph, i: (0, i, 0)),
            pl.BlockSpec((BR, H), lambda ph, i: (i, 0)),
            pl.BlockSpec((1, H), lambda ph, i: (0, 0)),
            pl.BlockSpec((1, H), lambda ph, i: (0, 0)),
            pl.BlockSpec((1, H), lambda ph, i: (0, 0)),
            pl.BlockSpec((BR, H), lambda ph, i: (i, 0)),
            pl.BlockSpec((BR, 1), lambda ph, i: (i, 0)),
            pl.BlockSpec((H, H), lambda ph, i: (0, 0)),
            pl.BlockSpec((1, H), lambda ph, i: (0, 0)),
            pl.BlockSpec((G, H), lambda ph, i: (0, 0)),
            pl.BlockSpec((1, G), lambda ph, i: (0, 0)),
            pl.BlockSpec((BR, 16), lambda ph, i: (i, 0)),
        ],
        out_specs=pl.BlockSpec((G, G), lambda ph, i: (0, 0)),
        out_shape=jax.ShapeDtypeStruct((G, G), jnp.float32),
        scratch_shapes=[
            pltpu.VMEM((NB, BR, H), jnp.float32),
            pltpu.VMEM((2, H), jnp.float32),
            pltpu.VMEM((G, H), jnp.float32),
            pltpu.VMEM((G, 8), jnp.float32),
        ],
    )(sacc, p, b, g, t, res, batch2d, lw0, lb0, lw1, lb1, dis16)


def kernel(x, edge_index, batch, W0, b0, W1, b1, W2, b2,
           g0, t0, g1, t1, g2, t2, lw0, lb0, lw1, lb1):
    src2d = edge_index[0].astype(jnp.int32).reshape(NC * NS, CPT, C)
    dst2d = edge_index[1].astype(jnp.int32).reshape(NC * NS, CPT, C)
    batch2d = batch.astype(jnp.int32).reshape(N, 1)

    b0r = b0.reshape(1, H); b1r = b1.reshape(1, H); b2r = b2.reshape(1, H)
    g0r = g0.reshape(1, H); g1r = g1.reshape(1, H); g2r = g2.reshape(1, H)
    t0r = t0.reshape(1, H); t1r = t1.reshape(1, H); t2r = t2.reshape(1, H)
    lb0r = lb0.reshape(1, H)
    lw1p = jnp.zeros((G, H), jnp.float32).at[0].set(lw1[0])
    lb1r = jnp.broadcast_to(lb1.reshape(1, 1), (1, G))

    dacc = _sc_degree(dst2d).reshape(NC, N, 16)
    p1, dis16 = _tc_first(x, W0, dacc)

    s1 = _sc_scatter(p1, src2d, dst2d).reshape(NC, N, H)
    h1, p2 = _tc_layer(s1, p1, b0r, g0r, t0r, None, W1, dis16)

    s2 = _sc_scatter(p2, src2d, dst2d).reshape(NC, N, H)
    h2, p3 = _tc_layer(s2, p2, b1r, g1r, t1r, h1, W2, dis16)

    s3 = _sc_scatter(p3, src2d, dst2d).reshape(NC, N, H)
    out = _tc_final(s3, p3, b2r, g2r, t2r, h2, batch2d,
                    lw0, lb0r, lw1p, lb1r, dis16)

    return out[:, 0]


# speedup vs baseline: 41.9274x; 1.1905x over previous
import functools

import jax
import jax.numpy as jnp
from jax import lax
from jax.experimental import pallas as pl
from jax.experimental.pallas import tpu as pltpu
from jax.experimental.pallas import tpu_sc as plsc

N = 10000
E = 320000
D_IN = 128
H = 64
G = 128

NC = 2
NS = 16
C = 80
CHUNKS = E // C
CPT = CHUNKS // (NC * NS)
RPW = N // NS

BR = 2000
NB = N // BR


def _sc_degree(dst2d):
    mesh = plsc.VectorSubcoreMesh(core_axis_name="c", subcore_axis_name="s")

    @functools.partial(
        pl.kernel,
        out_type=jax.ShapeDtypeStruct((NC, NS, RPW, 16), jnp.float32),
        mesh=mesh,
        scratch_types=[
            pltpu.VMEM((CPT, C), jnp.int32),
            pltpu.VMEM((C, 16), jnp.float32),
            pltpu.VMEM((RPW, 16), jnp.float32),
            pltpu.VMEM_SHARED((N, 16), jnp.float32),
            pltpu.SemaphoreType.DMA,
        ],
        compiler_params=pltpu.CompilerParams(use_tc_tiling_on_sc=False),
    )
    def body(dst_hbm, out_hbm, dstv, ones, wb, acc, sem):
        c = lax.axis_index("c")
        s = lax.axis_index("s")
        t = c * NS + s

        def fill(i, _):
            wb[i, :] = jnp.zeros((16,), jnp.float32)
            return 0
        lax.fori_loop(0, RPW, fill, 0)

        def fill1(i, _):
            ones[i, :] = jnp.ones((16,), jnp.float32)
            return 0
        lax.fori_loop(0, C, fill1, 0)

        pltpu.sync_copy(wb, acc.at[pl.ds(s * RPW, RPW)])
        plsc.subcore_barrier()

        pltpu.sync_copy(dst_hbm.at[t], dstv)

        def fire(j, _):
            pltpu.async_copy(ones, acc.at[dstv.at[j]], sem, add=True)
            return 0
        lax.fori_loop(0, CPT, fire, 0)

        def drain(j, _):
            pltpu.make_async_copy(ones, acc.at[dstv.at[j]], sem).wait()
            return 0
        lax.fori_loop(0, CPT, drain, 0)

        plsc.subcore_barrier()
        pltpu.sync_copy(acc.at[pl.ds(s * RPW, RPW)], wb)
        pltpu.sync_copy(wb, out_hbm.at[c, s])

    return body(dst2d)


def _sc_scatter(p, src2d, dst2d):
    mesh = plsc.VectorSubcoreMesh(core_axis_name="c", subcore_axis_name="s")

    SRO = 624
    SRN = 640

    @functools.partial(
        pl.kernel,
        out_type=jax.ShapeDtypeStruct((NC, NS, RPW, H), jnp.bfloat16),
        mesh=mesh,
        scratch_types=[
            pltpu.VMEM((CPT, C), jnp.int32),
            pltpu.VMEM((CPT, C), jnp.int32),
            [pltpu.VMEM((C, H), jnp.bfloat16) for _ in range(4)],
            pltpu.VMEM((SRN, H), jnp.bfloat16),
            pltpu.VMEM_SHARED((N, H), jnp.bfloat16),
            [pltpu.SemaphoreType.DMA for _ in range(4)],
        ],
        compiler_params=pltpu.CompilerParams(use_tc_tiling_on_sc=False),
    )
    def body(p_hbm, src_hbm, dst_hbm, out_hbm,
             srcv, dstv, bufs, wb, acc, sems):
        c = lax.axis_index("c")
        s = lax.axis_index("s")
        t = c * NS + s

        def fill(i, _):
            for j in range(H // 32):
                wb[i, pl.ds(j * 32, 32)] = jnp.zeros((32,), jnp.bfloat16)
            return 0
        lax.fori_loop(0, SRN, fill, 0)
        pltpu.sync_copy(wb.at[pl.ds(0, RPW)], acc.at[pl.ds(s * RPW, RPW)])
        plsc.subcore_barrier()

        pltpu.sync_copy(src_hbm.at[t], srcv)
        pltpu.sync_copy(dst_hbm.at[t], dstv)

        for b in range(4):
            pltpu.async_copy(p_hbm.at[srcv.at[b]], bufs[b], sems[b])

        def step(g, _):
            for b in range(4):
                jj = 4 * g + b
                pltpu.make_async_copy(p_hbm.at[srcv.at[jj]], bufs[b],
                                      sems[b]).wait()
                pltpu.sync_copy(bufs[b], acc.at[dstv.at[jj]], add=True)

                @pl.when(jj + 4 < CPT)
                def _():
                    pltpu.async_copy(p_hbm.at[srcv.at[jj + 4]], bufs[b],
                                     sems[b])
            return 0

        lax.fori_loop(0, CPT // 4, step, 0)
        pltpu.make_async_copy(p_hbm.at[srcv.at[CPT - 1]], bufs[0],
                              sems[0]).wait()
        pltpu.sync_copy(bufs[0], acc.at[dstv.at[CPT - 1]], add=True)

        plsc.subcore_barrier()
        pltpu.sync_copy(acc.at[pl.ds(s * RPW, RPW)], wb.at[pl.ds(0, RPW)])
        pltpu.sync_copy(wb.at[pl.ds(0, RPW)], out_hbm.at[c, s])

    return body(p, src2d, dst2d)


def _dot_t(a, w):
    return lax.dot_general(a, w, (((1,), (1,)), ((), ())),
                           preferred_element_type=jnp.float32)


def _tc_first(x, W0, dacc):
    def body(x_ref, w_ref, d_ref, p_ref, dis_ref):
        deg = d_ref[0] + d_ref[1] + 1.0
        dis = lax.rsqrt(deg)
        dis_ref[...] = dis
        p = _dot_t(x_ref[...], w_ref[...]) * dis[:, 0:1]
        p_ref[...] = p.astype(jnp.bfloat16)

    return pl.pallas_call(
        body,
        grid=(NB,),
        in_specs=[
            pl.BlockSpec((BR, D_IN), lambda i: (i, 0)),
            pl.BlockSpec((H, D_IN), lambda i: (0, 0)),
            pl.BlockSpec((NC, BR, 16), lambda i: (0, i, 0)),
        ],
        out_specs=[
            pl.BlockSpec((BR, H), lambda i: (i, 0)),
            pl.BlockSpec((BR, 16), lambda i: (i, 0)),
        ],
        out_shape=[
            jax.ShapeDtypeStruct((N, H), jnp.bfloat16),
            jax.ShapeDtypeStruct((N, 16), jnp.float32),
        ],
    )(x, W0, dacc)


def _tc_layer(sacc, p, b, g, t, res, Wn, dis16):
    has_res = res is not None

    def body(*refs):
        if has_res:
            (s_ref, p_ref, b_ref, g_ref, t_ref, r_ref, w_ref, dis_ref,
             h_ref, pn_ref, c_v, st_v) = refs
        else:
            (s_ref, p_ref, b_ref, g_ref, t_ref, w_ref, dis_ref,
             h_ref, pn_ref, c_v, st_v) = refs
        ph = pl.program_id(0)
        i = pl.program_id(1)

        @pl.when(ph == 0)
        def _():
            ssum = (s_ref[0].astype(jnp.float32) + s_ref[1].astype(jnp.float32)
                    + p_ref[...].astype(jnp.float32))
            cblk = dis_ref[:, 0:1] * ssum + b_ref[...]
            c_v[i] = cblk
            s1 = jnp.sum(cblk, axis=0, keepdims=True)
            s2 = jnp.sum(cblk * cblk, axis=0, keepdims=True)
            blk = jnp.concatenate([s1, s2], axis=0)

            @pl.when(i == 0)
            def _():
                st_v[...] = blk

            @pl.when(i > 0)
            def _():
                st_v[...] += blk

        @pl.when(ph == 1)
        def _():
            mu = st_v[0:1] / N
            var = st_v[1:2] / N - mu * mu
            scale = lax.rsqrt(var + 1e-5) * g_ref[...]
            h = jnp.maximum((c_v[i] - mu) * scale + t_ref[...], 0.0)
            if has_res:
                h = h + r_ref[...]
            h_ref[...] = h
            pn = _dot_t(h, w_ref[...]) * dis_ref[:, 0:1]
            pn_ref[...] = pn.astype(jnp.bfloat16)

    in_arrays = [sacc, p, b, g, t] + ([res] if has_res else []) + [Wn, dis16]
    in_specs = [
        pl.BlockSpec((NC, BR, H), lambda ph, i: (0, i, 0)),
        pl.BlockSpec((BR, H), lambda ph, i: (i, 0)),
        pl.BlockSpec((1, H), lambda ph, i: (0, 0)),
        pl.BlockSpec((1, H), lambda ph, i: (0, 0)),
        pl.BlockSpec((1, H), lambda ph, i: (0, 0)),
    ] + ([pl.BlockSpec((BR, H), lambda ph, i: (i, 0))] if has_res else []) + [
        pl.BlockSpec((H, H), lambda ph, i: (0, 0)),
        pl.BlockSpec((BR, 16), lambda ph, i: (i, 0)),
    ]
    return pl.pallas_call(
        body,
        grid=(2, NB),
        in_specs=in_specs,
        out_specs=[
            pl.BlockSpec((BR, H), lambda ph, i: (i, 0)),
            pl.BlockSpec((BR, H), lambda ph, i: (i, 0)),
        ],
        out_shape=[
            jax.ShapeDtypeStruct((N, H), jnp.float32),
            jax.ShapeDtypeStruct((N, H), jnp.bfloat16),
        ],
        scratch_shapes=[
            pltpu.VMEM((NB, BR, H), jnp.float32),
            pltpu.VMEM((2, H), jnp.float32),
        ],
    )(*in_arrays)


def _tc_final(sacc, p, b, g, t, res, batch2d, lw0, lb0, lw1, lb1, dis16):
    def body(s_ref, p_ref, bias_ref, g_ref, t_ref, r_ref, b_ref,
             w0_ref, b0_ref, w1_ref, b1_ref, dis_ref,
             o_ref, c_v, st_v, pool_acc, cnt_acc):
        ph = pl.program_id(0)
        i = pl.program_id(1)

        @pl.when(ph == 0)
        def _():
            ssump = (s_ref[0].astype(jnp.float32)
                     + s_ref[1].astype(jnp.float32)
                     + p_ref[...].astype(jnp.float32))
            cblk = dis_ref[:, 0:1] * ssump + bias_ref[...]
            c_v[i] = cblk
            ssum = jnp.sum(cblk, axis=0, keepdims=True)
            ssq = jnp.sum(cblk * cblk, axis=0, keepdims=True)
            blk = jnp.concatenate([ssum, ssq], axis=0)

            @pl.when(i == 0)
            def _():
                st_v[...] = blk

            @pl.when(i > 0)
            def _():
                st_v[...] += blk

        @pl.when(ph == 1)
        def _():
            mu = st_v[0:1] / N
            var = st_v[1:2] / N - mu * mu
            scale = lax.rsqrt(var + 1e-5) * g_ref[...]
            h = (jnp.maximum((c_v[i] - mu) * scale + t_ref[...], 0.0)
                 + r_ref[...])

            iota = lax.broadcasted_iota(jnp.int32, (1, G), 1)
            mask = (b_ref[...] == iota).astype(jnp.float32)
            pool_blk = lax.dot_general(mask, h, (((0,), (0,)), ((), ())),
                                       preferred_element_type=jnp.float32)
            cnt_blk = lax.dot_general(mask, jnp.ones((BR, 8), jnp.float32),
                                      (((0,), (0,)), ((), ())),
                                      preferred_element_type=jnp.float32)

            @pl.when(i == 0)
            def _():
                pool_acc[...] = pool_blk
                cnt_acc[...] = cnt_blk

            @pl.when(i > 0)
            def _():
                pool_acc[...] += pool_blk
                cnt_acc[...] += cnt_blk

            @pl.when(i == NB - 1)
            def _():
                cnt = jnp.maximum(cnt_acc[:, 0:1], 1.0)
                hp = pool_acc[...] / cnt
                z = jnp.maximum(_dot_t(hp, w0_ref[...]) + b0_ref[...], 0.0)
                o_ref[...] = _dot_t(z, w1_ref[...]) + b1_ref[...]

    return pl.pallas_call(
        body,
        grid=(2, NB),
        in_specs=[
            pl.BlockSpec((NC, BR, H), lambda ph, i: (0, i, 0)),
            pl.BlockSpec((BR, H), lambda ph, i: (i, 0)),
            pl.BlockSpec((1, H), lambda ph, i: (0, 0)),
            pl.BlockSpec((1, H), lambda ph, i: (0, 0)),
            pl.BlockSpec((1, H), lambda ph, i: (0, 0)),
            pl.BlockSpec((BR, H), lambda ph, i: (i, 0)),
            pl.BlockSpec((BR, 1), lambda ph, i: (i, 0)),
            pl.BlockSpec((H, H), lambda ph, i: (0, 0)),
            pl.BlockSpec((1, H), lambda ph, i: (0, 0)),
            pl.BlockSpec((G, H), lambda ph, i: (0, 0)),
            pl.BlockSpec((1, G), lambda ph, i: (0, 0)),
            pl.BlockSpec((BR, 16), lambda ph, i: (i, 0)),
        ],
        out_specs=pl.BlockSpec((G, G), lambda ph, i: (0, 0)),
        out_shape=jax.ShapeDtypeStruct((G, G), jnp.float32),
        scratch_shapes=[
            pltpu.VMEM((NB, BR, H), jnp.float32),
            pltpu.VMEM((2, H), jnp.float32),
            pltpu.VMEM((G, H), jnp.float32),
            pltpu.VMEM((G, 8), jnp.float32),
        ],
    )(sacc, p, b, g, t, res, batch2d, lw0, lb0, lw1, lb1, dis16)


def kernel(x, edge_index, batch, W0, b0, W1, b1, W2, b2,
           g0, t0, g1, t1, g2, t2, lw0, lb0, lw1, lb1):
    src2d = edge_index[0].astype(jnp.int32).reshape(NC * NS, CPT, C)
    dst2d = edge_index[1].astype(jnp.int32).reshape(NC * NS, CPT, C)
    batch2d = batch.astype(jnp.int32).reshape(N, 1)

    b0r = b0.reshape(1, H); b1r = b1.reshape(1, H); b2r = b2.reshape(1, H)
    g0r = g0.reshape(1, H); g1r = g1.reshape(1, H); g2r = g2.reshape(1, H)
    t0r = t0.reshape(1, H); t1r = t1.reshape(1, H); t2r = t2.reshape(1, H)
    lb0r = lb0.reshape(1, H)
    lw1p = jnp.zeros((G, H), jnp.float32).at[0].set(lw1[0])
    lb1r = jnp.broadcast_to(lb1.reshape(1, 1), (1, G))

    dacc = _sc_degree(dst2d).reshape(NC, N, 16)
    p1, dis16 = _tc_first(x, W0, dacc)

    s1 = _sc_scatter(p1, src2d, dst2d).reshape(NC, N, H)
    h1, p2 = _tc_layer(s1, p1, b0r, g0r, t0r, None, W1, dis16)

    s2 = _sc_scatter(p2, src2d, dst2d).reshape(NC, N, H)
    h2, p3 = _tc_layer(s2, p2, b1r, g1r, t1r, h1, W2, dis16)

    s3 = _sc_scatter(p3, src2d, dst2d).reshape(NC, N, H)
    out = _tc_final(s3, p3, b2r, g2r, t2r, h2, batch2d,
                    lw0, lb0r, lw1p, lb1r, dis16)

    return out[:, 0]
